# initial kernel scaffold (unmeasured)
import jax
import jax.numpy as jnp
from jax import lax
from jax.experimental import pallas as pl
from jax.experimental.pallas import tpu as pltpu

N_DEV = 32


def kernel(dy, W):
    m, _ = dy.shape
    n = W.shape[0]
    rows = m // N_DEV

    def body(dy_ref, w_ref, out_ref, p_ref, rs_buf, ag_buf,
             send1, recv1, send2, recv2):
        me = lax.axis_index("i")

        p_ref[:, :] = lax.dot_general(
            dy_ref[:, :], w_ref[:, :],
            dimension_numbers=(((1,), (1,)), ((), ())),
            preferred_element_type=jnp.float32,
        )

        sends1 = []
        for off in range(1, N_DEV):
            tgt = lax.rem(me + off, N_DEV)
            rdma = pltpu.make_async_remote_copy(
                src_ref=p_ref.at[pl.ds(tgt * rows, rows), :],
                dst_ref=rs_buf.at[me],
                send_sem=send1.at[off],
                recv_sem=recv1.at[me],
                device_id=tgt,
                device_id_type=pl.DeviceIdType.LOGICAL,
            )
            rdma.start()
            sends1.append(rdma)

        rs_buf[pl.ds(me, 1), :, :] = p_ref[
            pl.ds(me * rows, rows), :
        ].reshape(1, rows, n)

        for off in range(1, N_DEV):
            src = lax.rem(me + off, N_DEV)
            pltpu.make_async_remote_copy(
                src_ref=rs_buf.at[src],
                dst_ref=rs_buf.at[src],
                send_sem=send1.at[off],
                recv_sem=recv1.at[src],
                device_id=me,
                device_id_type=pl.DeviceIdType.LOGICAL,
            ).wait_recv()

        ag_buf[pl.ds(me, 1), :, :] = jnp.sum(
            rs_buf[:, :, :], axis=0, keepdims=True
        )

        sends2 = []
        for off in range(1, N_DEV):
            tgt = lax.rem(me + off, N_DEV)
            rdma = pltpu.make_async_remote_copy(
                src_ref=ag_buf.at[me],
                dst_ref=ag_buf.at[me],
                send_sem=send2.at[off],
                recv_sem=recv2.at[me],
                device_id=tgt,
                device_id_type=pl.DeviceIdType.LOGICAL,
            )
            rdma.start()
            sends2.append(rdma)

        for off in range(1, N_DEV):
            src = lax.rem(me + off, N_DEV)
            pltpu.make_async_remote_copy(
                src_ref=ag_buf.at[src],
                dst_ref=ag_buf.at[src],
                send_sem=send2.at[off],
                recv_sem=recv2.at[src],
                device_id=me,
                device_id_type=pl.DeviceIdType.LOGICAL,
            ).wait_recv()

        for rdma in sends1 + sends2:
            rdma.wait_send()

        for s in range(N_DEV):
            out_ref[pl.ds(s * rows, rows), :] = ag_buf[s, :, :]

    return pl.pallas_call(
        body,
        out_shape=jax.ShapeDtypeStruct((m, n), jnp.float32),
        in_specs=[
            pl.BlockSpec(memory_space=pltpu.VMEM),
            pl.BlockSpec(memory_space=pltpu.VMEM),
        ],
        out_specs=pl.BlockSpec(memory_space=pltpu.VMEM),
        scratch_shapes=[
            pltpu.VMEM((m, n), jnp.float32),
            pltpu.VMEM((N_DEV, rows, n), jnp.float32),
            pltpu.VMEM((N_DEV, rows, n), jnp.float32),
            pltpu.SemaphoreType.DMA((N_DEV,)),
            pltpu.SemaphoreType.DMA((N_DEV,)),
            pltpu.SemaphoreType.DMA((N_DEV,)),
            pltpu.SemaphoreType.DMA((N_DEV,)),
        ],
        compiler_params=pltpu.CompilerParams(collective_id=0),
    )(dy, W)


# baseline (device time: 16912 ns/iter reference)
import jax
import jax.numpy as jnp
from jax import lax
from jax.experimental import pallas as pl
from jax.experimental.pallas import tpu as pltpu

N_DEV = 32
P = 8
G = 4


def kernel(dy, W):
    m, _ = dy.shape
    n = W.shape[0]
    b1 = m // P
    b2 = b1 // G

    def body(dy_ref, w_ref, out_ref, p_ref, rs1_buf, red_ref, rs2_buf,
             ag2_buf, ag1_buf, send1, recv1, send2, recv2, send3, recv3,
             send4, recv4):
        me = lax.axis_index("i")
        g = lax.div(me, P)
        r = lax.rem(me, P)

        p_ref[:, :] = lax.dot_general(
            dy_ref[:, :], w_ref[:, :],
            dimension_numbers=(((1,), (1,)), ((), ())),
            preferred_element_type=jnp.float32,
        )

        sends = []

        for off in range(1, P):
            jr = lax.rem(r + off, P)
            rdma = pltpu.make_async_remote_copy(
                src_ref=p_ref.at[pl.ds(jr * b1, b1), :],
                dst_ref=rs1_buf.at[r],
                send_sem=send1.at[off],
                recv_sem=recv1.at[r],
                device_id=g * P + jr,
                device_id_type=pl.DeviceIdType.LOGICAL,
            )
            rdma.start()
            sends.append(rdma)
        rs1_buf[pl.ds(r, 1), :, :] = p_ref[pl.ds(r * b1, b1), :].reshape(
            1, b1, n
        )
        for off in range(1, P):
            s = lax.rem(r + off, P)
            pltpu.make_async_remote_copy(
                src_ref=rs1_buf.at[s],
                dst_ref=rs1_buf.at[s],
                send_sem=send1.at[off],
                recv_sem=recv1.at[s],
                device_id=me,
                device_id_type=pl.DeviceIdType.LOGICAL,
            ).wait_recv()
        red_ref[:, :] = jnp.sum(rs1_buf[:, :, :], axis=0)

        for off in range(1, G):
            qg = lax.rem(g + off, G)
            rdma = pltpu.make_async_remote_copy(
                src_ref=red_ref.at[pl.ds(qg * b2, b2), :],
                dst_ref=rs2_buf.at[g],
                send_sem=send2.at[off],
                recv_sem=recv2.at[g],
                device_id=qg * P + r,
                device_id_type=pl.DeviceIdType.LOGICAL,
            )
            rdma.start()
            sends.append(rdma)
        rs2_buf[pl.ds(g, 1), :, :] = red_ref[pl.ds(g * b2, b2), :].reshape(
            1, b2, n
        )
        for off in range(1, G):
            s = lax.rem(g + off, G)
            pltpu.make_async_remote_copy(
                src_ref=rs2_buf.at[s],
                dst_ref=rs2_buf.at[s],
                send_sem=send2.at[off],
                recv_sem=recv2.at[s],
                device_id=me,
                device_id_type=pl.DeviceIdType.LOGICAL,
            ).wait_recv()
        ag2_buf[pl.ds(g, 1), :, :] = jnp.sum(
            rs2_buf[:, :, :], axis=0, keepdims=True
        )

        for off in range(1, G):
            qg = lax.rem(g + off, G)
            rdma = pltpu.make_async_remote_copy(
                src_ref=ag2_buf.at[g],
                dst_ref=ag2_buf.at[g],
                send_sem=send3.at[off],
                recv_sem=recv3.at[g],
                device_id=qg * P + r,
                device_id_type=pl.DeviceIdType.LOGICAL,
            )
            rdma.start()
            sends.append(rdma)
        for off in range(1, G):
            s = lax.rem(g + off, G)
            pltpu.make_async_remote_copy(
                src_ref=ag2_buf.at[s],
                dst_ref=ag2_buf.at[s],
                send_sem=send3.at[off],
                recv_sem=recv3.at[s],
                device_id=me,
                device_id_type=pl.DeviceIdType.LOGICAL,
            ).wait_recv()
        ag1_buf[pl.ds(r, 1), :, :] = ag2_buf[:, :, :].reshape(1, b1, n)

        for off in range(1, P):
            jr = lax.rem(r + off, P)
            rdma = pltpu.make_async_remote_copy(
                src_ref=ag1_buf.at[r],
                dst_ref=ag1_buf.at[r],
                send_sem=send4.at[off],
                recv_sem=recv4.at[r],
                device_id=g * P + jr,
                device_id_type=pl.DeviceIdType.LOGICAL,
            )
            rdma.start()
            sends.append(rdma)
        for off in range(1, P):
            s = lax.rem(r + off, P)
            pltpu.make_async_remote_copy(
                src_ref=ag1_buf.at[s],
                dst_ref=ag1_buf.at[s],
                send_sem=send4.at[off],
                recv_sem=recv4.at[s],
                device_id=me,
                device_id_type=pl.DeviceIdType.LOGICAL,
            ).wait_recv()

        for rdma in sends:
            rdma.wait_send()

        for s in range(P):
            out_ref[pl.ds(s * b1, b1), :] = ag1_buf[s, :, :]

    return pl.pallas_call(
        body,
        out_shape=jax.ShapeDtypeStruct((m, n), jnp.float32),
        in_specs=[
            pl.BlockSpec(memory_space=pltpu.VMEM),
            pl.BlockSpec(memory_space=pltpu.VMEM),
        ],
        out_specs=pl.BlockSpec(memory_space=pltpu.VMEM),
        scratch_shapes=[
            pltpu.VMEM((m, n), jnp.float32),
            pltpu.VMEM((P, b1, n), jnp.float32),
            pltpu.VMEM((b1, n), jnp.float32),
            pltpu.VMEM((G, b2, n), jnp.float32),
            pltpu.VMEM((G, b2, n), jnp.float32),
            pltpu.VMEM((P, b1, n), jnp.float32),
            pltpu.SemaphoreType.DMA((P,)),
            pltpu.SemaphoreType.DMA((P,)),
            pltpu.SemaphoreType.DMA((G,)),
            pltpu.SemaphoreType.DMA((G,)),
            pltpu.SemaphoreType.DMA((G,)),
            pltpu.SemaphoreType.DMA((G,)),
            pltpu.SemaphoreType.DMA((P,)),
            pltpu.SemaphoreType.DMA((P,)),
        ],
    )(dy, W)
